# baseline (speedup 1.0000x reference)
import functools

import jax
import jax.numpy as jnp
from jax import lax
from jax.experimental import pallas as pl
from jax.experimental.pallas import tpu as pltpu
from jax.experimental.pallas import tpu_sc as plsc

POI = 1000000
D = 64
B = 4096
H = 200
CH = 128
NBUF = 4

NC = 2
NS = 16
NW = NC * NS
NG = H // NBUF


@functools.lru_cache(maxsize=1)
def _build():
    mesh = plsc.VectorSubcoreMesh(core_axis_name="c", subcore_axis_name="s")

    @functools.partial(
        pl.kernel,
        mesh=mesh,
        out_type=jax.ShapeDtypeStruct((H, D, B), jnp.float32),
        compiler_params=pltpu.CompilerParams(
            use_tc_tiling_on_sc=True, needs_layout_passes=False),
        scratch_types=(
            [pltpu.VMEM((H, CH), jnp.int32)]
            + [pltpu.VMEM((CH,), jnp.int32) for _ in range(NBUF)]
            + [pltpu.VMEM((CH, 128), jnp.float32) for _ in range(NBUF)]
            + [pltpu.VMEM((D, CH), jnp.float32) for _ in range(NBUF)]
            + [pltpu.SemaphoreType.DMA for _ in range(2 * NBUF)]
        ),
    )
    def gather_kernel(tbl_hbm, idxT_hbm, out_hbm, idx_v, *rest):
        pair = rest[:NBUF]
        g = rest[NBUF:2 * NBUF]
        o = rest[2 * NBUF:3 * NBUF]
        gsem = rest[3 * NBUF:4 * NBUF]
        wsem = rest[4 * NBUF:]

        wid = lax.axis_index("s") * NC + lax.axis_index("c")
        b0 = wid * CH
        pltpu.sync_copy(idxT_hbm.at[:, pl.ds(b0, CH)], idx_v)

        iota = lax.iota(jnp.int32, 16)

        def prep(j, b):
            for kk in range(8):
                iv = idx_v[j, pl.ds(16 * kk, 16)]
                pair[b][pl.ds(16 * kk, 16)] = iv >> 1

        def compute(j, b):
            cvecs = [iota + (16 * kk) for kk in range(8)]
            cols0 = []
            for kk in range(8):
                iv = idx_v[j, pl.ds(16 * kk, 16)]
                cols0.append((iv & 1) * 64)

            def row(j2, cols):
                for kk in range(8):
                    vals = plsc.load_gather(g[b], [cvecs[kk], cols[kk]])
                    o[b][j2, pl.ds(16 * kk, 16)] = vals
                return tuple(c + 1 for c in cols)

            lax.fori_loop(0, D, row, tuple(cols0))

        for b in range(NBUF):
            prep(b, b)
            pltpu.async_copy(tbl_hbm.at[pair[b]], g[b], gsem[b])

        def group(grp, carry):
            for b in range(NBUF):
                j = grp * NBUF + b
                pltpu.make_async_copy(tbl_hbm.at[pair[b]], g[b], gsem[b]).wait()

                @pl.when(grp > 0)
                def _():
                    pltpu.make_async_copy(
                        o[b], out_hbm.at[0, :, pl.ds(0, CH)], wsem[b]).wait()

                compute(j, b)

                @pl.when(grp < NG - 1)
                def _():
                    prep(j + NBUF, b)
                    pltpu.async_copy(tbl_hbm.at[pair[b]], g[b], gsem[b])

                pltpu.async_copy(o[b], out_hbm.at[j, :, pl.ds(b0, CH)], wsem[b])
            return carry

        lax.fori_loop(0, NG, group, 0)

        for b in range(NBUF):
            pltpu.make_async_copy(
                o[b], out_hbm.at[0, :, pl.ds(0, CH)], wsem[b]).wait()

    return gather_kernel


def kernel(traj, table):
    tbl2 = table.reshape(POI // 2, 128)
    trajT = traj.T.astype(jnp.int32)
    out3 = _build()(tbl2, trajT)
    return out3.transpose(2, 0, 1)


# speedup vs baseline: 1.4811x; 1.4811x over previous
import functools

import jax
import jax.numpy as jnp
from jax import lax
from jax.experimental import pallas as pl
from jax.experimental.pallas import tpu as pltpu
from jax.experimental.pallas import tpu_sc as plsc

POI = 1000000
D = 64
B = 4096
H = 200
HE = 104
HO = 96
NBUF = 4

NC = 2
NS = 16
NW = NC * NS
BPW = B // NW
NCH = BPW * 2
NG = NCH // NBUF


@functools.lru_cache(maxsize=1)
def _build():
    mesh = plsc.VectorSubcoreMesh(core_axis_name="c", subcore_axis_name="s")

    @functools.partial(
        pl.kernel,
        mesh=mesh,
        out_type=jax.ShapeDtypeStruct((B, H, D), jnp.float32),
        compiler_params=pltpu.CompilerParams(use_tc_tiling_on_sc=False),
        scratch_types=(
            [pltpu.VMEM((BPW, HE), jnp.int32), pltpu.VMEM((BPW, HO), jnp.int32)]
            + [pltpu.VMEM(((HE, D) if b % 2 == 0 else (HO, D)), jnp.float32)
               for b in range(NBUF)]
            + [pltpu.SemaphoreType.DMA for _ in range(2 * NBUF)]
        ),
    )
    def gather_kernel(tbl_hbm, traj_hbm, out_hbm, idx_e, idx_o, *rest):
        rows = rest[:NBUF]
        gsem = rest[NBUF:2 * NBUF]
        wsem = rest[2 * NBUF:]

        wid = lax.axis_index("s") * NC + lax.axis_index("c")
        b0 = wid * BPW
        pltpu.sync_copy(traj_hbm.at[pl.ds(b0, BPW), pl.ds(0, HE)], idx_e)
        pltpu.sync_copy(traj_hbm.at[pl.ds(b0, BPW), pl.ds(HE, HO)], idx_o)

        for b in range(NBUF):
            bb, half = b // 2, b % 2
            src = idx_o if half else idx_e
            pltpu.async_copy(tbl_hbm.at[src.at[bb]], rows[b], gsem[b])

        def group(grp, carry):
            for b in range(NBUF):
                bb = grp * 2 + b // 2
                half = b % 2
                src = idx_o if half else idx_e
                pltpu.make_async_copy(
                    tbl_hbm.at[src.at[0]], rows[b], gsem[b]).wait()
                h0, hn = (HE, HO) if half else (0, HE)
                pltpu.async_copy(
                    rows[b],
                    out_hbm.at[b0 + bb, pl.ds(h0, hn), :],
                    wsem[b])

                @pl.when(grp < NG - 1)
                def _():
                    pltpu.make_async_copy(
                        rows[b], out_hbm.at[0, pl.ds(0, hn), :],
                        wsem[b]).wait()
                    pltpu.async_copy(
                        tbl_hbm.at[src.at[bb + 2]], rows[b], gsem[b])
            return carry

        lax.fori_loop(0, NG, group, 0)

        for b in range(NBUF):
            hn = HE if b % 2 == 0 else HO
            pltpu.make_async_copy(
                rows[b], out_hbm.at[0, pl.ds(0, hn), :], wsem[b]).wait()

    return gather_kernel


def kernel(traj, table):
    return _build()(table, traj.astype(jnp.int32))


# speedup vs baseline: 1.5607x; 1.0537x over previous
import functools

import jax
import jax.numpy as jnp
from jax import lax
from jax.experimental import pallas as pl
from jax.experimental.pallas import tpu as pltpu
from jax.experimental.pallas import tpu_sc as plsc

POI = 1000000
D = 64
B = 4096
H = 200
TOT = B * H
CH = 128
NBUF = 4

NC = 2
NS = 16
NW = NC * NS
NCH = TOT // (NW * CH)
NG = NCH // NBUF


@functools.lru_cache(maxsize=1)
def _build_gather():
    mesh = plsc.VectorSubcoreMesh(core_axis_name="c", subcore_axis_name="s")

    @functools.partial(
        pl.kernel,
        mesh=mesh,
        out_type=jax.ShapeDtypeStruct((TOT, D), jnp.float32),
        compiler_params=pltpu.CompilerParams(use_tc_tiling_on_sc=False),
        scratch_types=(
            [pltpu.VMEM((NCH, CH), jnp.int32)]
            + [pltpu.VMEM((CH, D), jnp.float32) for _ in range(NBUF)]
            + [pltpu.SemaphoreType.DMA for _ in range(2 * NBUF)]
        ),
    )
    def gather_kernel(table_hbm, idx_hbm, out_hbm, idx_v, *rest):
        rows = rest[:NBUF]
        gsem = rest[NBUF:2 * NBUF]
        wsem = rest[2 * NBUF:]

        wid = lax.axis_index("s") * NC + lax.axis_index("c")
        pltpu.sync_copy(idx_hbm.at[pl.ds(wid * NCH, NCH)], idx_v)
        out_base = wid * NCH * CH

        for b in range(NBUF):
            pltpu.async_copy(table_hbm.at[idx_v.at[b]], rows[b], gsem[b])

        def group(g, carry):
            for b in range(NBUF):
                j = g * NBUF + b
                pltpu.make_async_copy(
                    table_hbm.at[idx_v.at[b]], rows[b], gsem[b]).wait()
                row0 = pl.multiple_of(out_base + j * CH, CH)
                pltpu.async_copy(
                    rows[b], out_hbm.at[pl.ds(row0, CH)], wsem[b])

                @pl.when(g < NG - 1)
                def _():
                    pltpu.make_async_copy(
                        rows[b], out_hbm.at[pl.ds(0, CH)], wsem[b]).wait()
                    pltpu.async_copy(
                        table_hbm.at[idx_v.at[j + NBUF]], rows[b], gsem[b])
            return carry

        lax.fori_loop(0, NG, group, 0)

        for b in range(NBUF):
            pltpu.make_async_copy(
                rows[b], out_hbm.at[pl.ds(0, CH)], wsem[b]).wait()

    return gather_kernel


BB = 128


def _transpose_block(x_ref, o_ref):
    for hh in range(H // 2):
        x = x_ref[:, hh, :]
        o_ref[2 * hh] = x[:, :D].T
        o_ref[2 * hh + 1] = x[:, D:].T


@functools.lru_cache(maxsize=1)
def _build_transpose():
    return pl.pallas_call(
        _transpose_block,
        grid=(B // BB,),
        in_specs=[pl.BlockSpec((BB, H // 2, 2 * D), lambda bi: (bi, 0, 0))],
        out_specs=pl.BlockSpec((H, D, BB), lambda bi: (0, 0, bi)),
        out_shape=jax.ShapeDtypeStruct((H, D, B), jnp.float32),
    )


def kernel(traj, table):
    flat_idx = traj.reshape(TOT // CH, CH).astype(jnp.int32)
    flat = _build_gather()(table, flat_idx)
    flat3 = flat.reshape(B, H // 2, 2 * D)
    out3 = _build_transpose()(flat3)
    return out3.transpose(2, 0, 1)
